# baseline (speedup 1.0000x reference)
import functools

import jax
import jax.numpy as jnp
from jax import lax
from jax.experimental import pallas as pl
from jax.experimental.pallas import tpu as pltpu
from jax.experimental.pallas import tpu_sc as plsc

_NC = 2
_NS = 16
_NW = _NC * _NS
_L = 16
_CH = 128


def _sc_counts(src_ids, dst_ids, t_flat, table):
    b = src_ids.shape[0]
    bpw = b // _NW
    nchunk = bpw // _CH
    gpc = _CH // _L
    max_deg = 64

    mesh = plsc.VectorSubcoreMesh(core_axis_name="c", subcore_axis_name="s")

    @functools.partial(
        pl.kernel,
        mesh=mesh,
        out_type=[jax.ShapeDtypeStruct((b,), jnp.float32) for _ in range(3)],
        compiler_params=pltpu.CompilerParams(
            needs_layout_passes=False, use_tc_tiling_on_sc=True),
        scratch_types=[
            pltpu.VMEM((2, _CH), jnp.int32),
            pltpu.VMEM((2, _CH), jnp.int32),
            pltpu.VMEM((bpw,), jnp.float32),
            pltpu.VMEM((2, _CH, 128), jnp.float32),
            pltpu.VMEM((2, _CH, 128), jnp.float32),
            pltpu.VMEM((bpw,), jnp.float32),
            pltpu.VMEM((bpw,), jnp.float32),
            pltpu.VMEM((bpw,), jnp.float32),
            pltpu.SemaphoreType.DMA,
            pltpu.SemaphoreType.DMA,
            pltpu.SemaphoreType.DMA,
            pltpu.SemaphoreType.DMA,
        ],
    )
    def body(src_hbm, dst_hbm, t_hbm, tab_hbm,
             o1_hbm, o2_hbm, o3_hbm,
             idx_s, idx_d, t_v, rs, rd, o1, o2, o3, *sems):
        wid = lax.axis_index("s") * _NC + lax.axis_index("c")
        base = wid * bpw
        pltpu.sync_copy(t_hbm.at[pl.ds(base, bpw)], t_v)

        def issue(j):
            p = j % 2
            sl = pl.ds(base + j * _CH, _CH)
            pltpu.sync_copy(src_hbm.at[sl], idx_s.at[p])
            pltpu.sync_copy(dst_hbm.at[sl], idx_d.at[p])
            return (
                pltpu.async_copy(tab_hbm.at[idx_s.at[p]], rs.at[p], sems[p]),
                pltpu.async_copy(tab_hbm.at[idx_d.at[p]], rd.at[p], sems[2 + p]),
            )

        pending = {0: issue(0)}
        for j in range(nchunk):
            p = j % 2
            if j + 1 < nchunk:
                pending[j + 1] = issue(j + 1)
            for c in pending.pop(j):
                c.wait()
            rs_p, rd_p = rs.at[p], rd.at[p]
            idx_sp = idx_s.at[p]

            def grp(gg, carry, j=j, rs_p=rs_p, rd_p=rd_p, idx_sp=idx_sp):
                osl = pl.ds(j * _CH + gg * _L, _L)
                t16 = t_v[osl]
                s16 = idx_sp[pl.ds(gg * _L, _L)].astype(jnp.float32)
                row = gg * _L + lax.iota(jnp.int32, _L)

                def lower_bound(ref):
                    pos = jnp.zeros((_L,), jnp.int32)
                    for s in (32, 16, 8, 4, 2, 1):
                        cand = pos + (s - 1)
                        v = plsc.load_gather(ref, [row, cand])
                        pos = pos + jnp.where(v < t16, s, 0)
                    v = plsc.load_gather(ref, [row, pos])
                    return pos + jnp.where(v < t16, 1, 0)

                pos1 = lower_bound(rs_p)
                pos2 = lower_bound(rd_p)
                a3 = jnp.zeros((_L,), jnp.int32)
                one = jnp.ones((_L,), jnp.int32)
                zero = jnp.zeros((_L,), jnp.int32)
                for d in range(max_deg):
                    dd = jnp.full((_L,), 64 + d, jnp.int32)
                    dn = plsc.load_gather(rd_p, [row, dd])
                    c3 = jnp.logical_and(dn == s16, pos2 > dd - 64)
                    a3 = a3 + jnp.where(c3, one, zero)
                o1[osl] = pos1.astype(jnp.float32)
                o2[osl] = pos2.astype(jnp.float32)
                o3[osl] = a3.astype(jnp.float32)
                return carry

            lax.fori_loop(0, gpc, grp, 0)

        pltpu.sync_copy(o1, o1_hbm.at[pl.ds(base, bpw)])
        pltpu.sync_copy(o2, o2_hbm.at[pl.ds(base, bpw)])
        pltpu.sync_copy(o3, o3_hbm.at[pl.ds(base, bpw)])

    return body(src_ids, dst_ids, t_flat, table)


def _merge_tables(nt_t, nid_t, eye):
    depth, n = nt_t.shape
    blkn = 16384
    grid = (pl.cdiv(n, blkn),)

    def body(a, bb, ee, out):
        del ee
        out[:, 0:depth] = a[:].T
        out[:, depth:2 * depth] = bb[:].T.astype(jnp.float32)

    return pl.pallas_call(
        body,
        grid=grid,
        in_specs=[
            pl.BlockSpec((depth, blkn), lambda i: (0, i)),
            pl.BlockSpec((depth, blkn), lambda i: (0, i)),
            pl.BlockSpec((depth, depth), lambda i: (0, 0)),
        ],
        out_specs=pl.BlockSpec((blkn, 2 * depth), lambda i: (i, 0)),
        out_shape=jax.ShapeDtypeStruct((n, 2 * depth), jnp.float32),
    )(nt_t, nid_t, eye)


def _mlp(i1_t, i2_t, f1, f2, f3, Wf1, bf1, W2c, W1ab, b1p, W2, b2):
    in_dim, b = i1_t.shape
    out_dim = W2.shape[1]
    blk = 4096
    grid = (b // blk,)
    cdims = (((0,), (0,)), ((), ()))

    def body(i1, i2, f1r, f2r, f3r, wf1, bf1r, w2c, w1ab, b1r, w2, b2r, out):
        fmat = jnp.concatenate(
            [f1r[:][None, :], f2r[:][None, :], f3r[:][None, :]], axis=0)
        z = lax.dot_general(fmat, wf1[:], cdims)
        ff = jax.nn.relu(z + bf1r[:][None, :])
        h = (lax.dot_general(i1[:], w1ab[0:in_dim, :], cdims)
             + lax.dot_general(i2[:], w1ab[in_dim:2 * in_dim, :], cdims)
             + ff @ w2c[:]
             + b1r[:][None, :])
        out[:] = (lax.dot_general(w2[:], jax.nn.relu(h), (((0,), (1,)), ((), ())))
                  + b2r[:][:, None])

    full2 = lambda shape: pl.BlockSpec(shape, lambda i: (0, 0))
    full1 = lambda shape: pl.BlockSpec(shape, lambda i: (0,))
    return pl.pallas_call(
        body,
        grid=grid,
        in_specs=[
            pl.BlockSpec((in_dim, blk), lambda i: (0, i)),
            pl.BlockSpec((in_dim, blk), lambda i: (0, i)),
            pl.BlockSpec((blk,), lambda i: (i,)),
            pl.BlockSpec((blk,), lambda i: (i,)),
            pl.BlockSpec((blk,), lambda i: (i,)),
            full2(Wf1.shape),
            full1(bf1.shape),
            full2(W2c.shape),
            full2(W1ab.shape),
            full1(b1p.shape),
            full2(W2.shape),
            full1(b2.shape),
        ],
        out_specs=pl.BlockSpec((out_dim, blk), lambda i: (0, i)),
        out_shape=jax.ShapeDtypeStruct((out_dim, b), jnp.float32),
    )(i1_t, i2_t, f1, f2, f3, Wf1, bf1, W2c, W1ab, b1p, W2, b2)


def kernel(input_1, input_2, src_ids, dst_ids, interact_times, neighbor_times,
           neighbor_ids, Wf1, bf1, Wf2, bf2, W1, b1, W2, b2):
    in_dim = input_1.shape[1]
    eye = jnp.eye(neighbor_times.shape[1], dtype=jnp.float32)
    table = _merge_tables(neighbor_times.T, neighbor_ids.astype(jnp.int32).T,
                          eye)
    f1, f2, f3 = _sc_counts(src_ids.astype(jnp.int32),
                            dst_ids.astype(jnp.int32), interact_times, table)
    W1c = W1[2 * in_dim:3 * in_dim, :]
    W2c = Wf2 @ W1c
    b1p = b1 + bf2 @ W1c
    W1ab = W1[0:2 * in_dim, :]
    return _mlp(input_1.T, input_2.T, f1, f2, f3, Wf1, bf1, W2c, W1ab, b1p,
                W2, b2).T


# speedup vs baseline: 1.0570x; 1.0570x over previous
import functools

import jax
import jax.numpy as jnp
from jax import lax
from jax.experimental import pallas as pl
from jax.experimental.pallas import tpu as pltpu
from jax.experimental.pallas import tpu_sc as plsc

_NC = 2
_NS = 16
_NW = _NC * _NS
_L = 16
_CH = 128


def _sc_counts(src_ids, dst_ids, t_flat, table):
    b = src_ids.shape[0]
    bpw = b // _NW
    nchunk = bpw // _CH
    gpc = _CH // _L
    max_deg = 64

    mesh = plsc.VectorSubcoreMesh(core_axis_name="c", subcore_axis_name="s")

    @functools.partial(
        pl.kernel,
        mesh=mesh,
        out_type=[jax.ShapeDtypeStruct((b,), jnp.float32) for _ in range(3)],
        compiler_params=pltpu.CompilerParams(
            needs_layout_passes=False, use_tc_tiling_on_sc=True),
        scratch_types=[
            pltpu.VMEM((2, _CH), jnp.int32),
            pltpu.VMEM((2, _CH), jnp.int32),
            pltpu.VMEM((bpw,), jnp.float32),
            pltpu.VMEM((2, _CH, 128), jnp.float32),
            pltpu.VMEM((2, _CH, 128), jnp.float32),
            pltpu.VMEM((bpw,), jnp.float32),
            pltpu.VMEM((bpw,), jnp.float32),
            pltpu.VMEM((bpw,), jnp.float32),
            pltpu.SemaphoreType.DMA,
            pltpu.SemaphoreType.DMA,
            pltpu.SemaphoreType.DMA,
            pltpu.SemaphoreType.DMA,
        ],
    )
    def body(src_hbm, dst_hbm, t_hbm, tab_hbm,
             o1_hbm, o2_hbm, o3_hbm,
             idx_s, idx_d, t_v, rs, rd, o1, o2, o3, *sems):
        wid = lax.axis_index("s") * _NC + lax.axis_index("c")
        base = wid * bpw
        pltpu.sync_copy(t_hbm.at[pl.ds(base, bpw)], t_v)

        def issue(j):
            p = j % 2
            sl = pl.ds(base + j * _CH, _CH)
            pltpu.sync_copy(src_hbm.at[sl], idx_s.at[p])
            pltpu.sync_copy(dst_hbm.at[sl], idx_d.at[p])
            return (
                pltpu.async_copy(tab_hbm.at[idx_s.at[p]], rs.at[p], sems[p]),
                pltpu.async_copy(tab_hbm.at[idx_d.at[p]], rd.at[p], sems[2 + p]),
            )

        pending = {0: issue(0)}
        for j in range(nchunk):
            p = j % 2
            if j + 1 < nchunk:
                pending[j + 1] = issue(j + 1)
            for c in pending.pop(j):
                c.wait()
            rs_p, rd_p = rs.at[p], rd.at[p]
            idx_sp = idx_s.at[p]

            def grp(gg, carry, j=j, rs_p=rs_p, rd_p=rd_p, idx_sp=idx_sp):
                osl = pl.ds(j * _CH + gg * _L, _L)
                t16 = t_v[osl]
                s16 = idx_sp[pl.ds(gg * _L, _L)].astype(jnp.float32)
                row = gg * _L + lax.iota(jnp.int32, _L)

                def lower_bound(ref):
                    pos = jnp.zeros((_L,), jnp.int32)
                    for s in (32, 16, 8, 4, 2, 1):
                        cand = pos + (s - 1)
                        v = plsc.load_gather(ref, [row, cand])
                        pos = pos + jnp.where(v < t16, s, 0)
                    v = plsc.load_gather(ref, [row, pos])
                    return pos + jnp.where(v < t16, 1, 0)

                pos1 = lower_bound(rs_p)
                pos2 = lower_bound(rd_p)
                a3 = jnp.zeros((_L,), jnp.int32)
                one = jnp.ones((_L,), jnp.int32)
                zero = jnp.zeros((_L,), jnp.int32)
                for d in range(max_deg):
                    dd = jnp.full((_L,), 64 + d, jnp.int32)
                    dn = plsc.load_gather(rd_p, [row, dd])
                    c3 = jnp.logical_and(dn == s16, pos2 > dd - 64)
                    a3 = a3 + jnp.where(c3, one, zero)
                o1[osl] = pos1.astype(jnp.float32)
                o2[osl] = pos2.astype(jnp.float32)
                o3[osl] = a3.astype(jnp.float32)
                return carry

            lax.fori_loop(0, gpc, grp, 0)

        pltpu.sync_copy(o1, o1_hbm.at[pl.ds(base, bpw)])
        pltpu.sync_copy(o2, o2_hbm.at[pl.ds(base, bpw)])
        pltpu.sync_copy(o3, o3_hbm.at[pl.ds(base, bpw)])

    return body(src_ids, dst_ids, t_flat, table)


def _merge_tables(nt_t, nid_t, eye):
    depth, n = nt_t.shape
    blkn = 8192
    grid = (pl.cdiv(n, blkn),)

    def body(a, bb, ee, out):
        del ee
        out[:, 0:depth] = a[:].T
        out[:, depth:2 * depth] = bb[:].T.astype(jnp.float32)

    return pl.pallas_call(
        body,
        grid=grid,
        in_specs=[
            pl.BlockSpec((depth, blkn), lambda i: (0, i)),
            pl.BlockSpec((depth, blkn), lambda i: (0, i)),
            pl.BlockSpec((depth, depth), lambda i: (0, 0)),
        ],
        out_specs=pl.BlockSpec((blkn, 2 * depth), lambda i: (i, 0)),
        out_shape=jax.ShapeDtypeStruct((n, 2 * depth), jnp.float32),
    )(nt_t, nid_t, eye)


def _mlp_a(i1_t, i2_t, W1ab, b1p):
    in_dim, b = i1_t.shape
    blk = 4096
    grid = (b // blk,)
    cdims = (((0,), (0,)), ((), ()))

    def body(i1, i2, w1ab, b1r, out):
        out[:] = (lax.dot_general(i1[:], w1ab[0:in_dim, :], cdims)
                  + lax.dot_general(i2[:], w1ab[in_dim:2 * in_dim, :], cdims)
                  + b1r[:][None, :])

    return pl.pallas_call(
        body,
        grid=grid,
        in_specs=[
            pl.BlockSpec((in_dim, blk), lambda i: (0, i)),
            pl.BlockSpec((in_dim, blk), lambda i: (0, i)),
            pl.BlockSpec(W1ab.shape, lambda i: (0, 0)),
            pl.BlockSpec(b1p.shape, lambda i: (0,)),
        ],
        out_specs=pl.BlockSpec((blk, in_dim), lambda i: (i, 0)),
        out_shape=jax.ShapeDtypeStruct((b, in_dim), jnp.float32),
    )(i1_t, i2_t, W1ab, b1p)


def _mlp_b(hpart, f1, f2, f3, Wf1, bf1, W2c, W2, b2):
    b, hid = hpart.shape
    out_dim = W2.shape[1]
    blk = 4096
    grid = (b // blk,)
    cdims = (((0,), (0,)), ((), ()))

    def body(hp, f1r, f2r, f3r, wf1, bf1r, w2c, w2, b2r, out):
        fmat = jnp.concatenate(
            [f1r[:][None, :], f2r[:][None, :], f3r[:][None, :]], axis=0)
        z = lax.dot_general(fmat, wf1[:], cdims)
        ff = jax.nn.relu(z + bf1r[:][None, :])
        h = hp[:] + ff @ w2c[:]
        out[:] = (lax.dot_general(w2[:], jax.nn.relu(h), (((0,), (1,)), ((), ())))
                  + b2r[:][:, None])

    full2 = lambda shape: pl.BlockSpec(shape, lambda i: (0, 0))
    full1 = lambda shape: pl.BlockSpec(shape, lambda i: (0,))
    return pl.pallas_call(
        body,
        grid=grid,
        in_specs=[
            pl.BlockSpec((blk, hid), lambda i: (i, 0)),
            pl.BlockSpec((blk,), lambda i: (i,)),
            pl.BlockSpec((blk,), lambda i: (i,)),
            pl.BlockSpec((blk,), lambda i: (i,)),
            full2(Wf1.shape),
            full1(bf1.shape),
            full2(W2c.shape),
            full2(W2.shape),
            full1(b2.shape),
        ],
        out_specs=pl.BlockSpec((out_dim, blk), lambda i: (0, i)),
        out_shape=jax.ShapeDtypeStruct((out_dim, b), jnp.float32),
    )(hpart, f1, f2, f3, Wf1, bf1, W2c, W2, b2)


def kernel(input_1, input_2, src_ids, dst_ids, interact_times, neighbor_times,
           neighbor_ids, Wf1, bf1, Wf2, bf2, W1, b1, W2, b2):
    in_dim = input_1.shape[1]
    eye = jnp.eye(neighbor_times.shape[1], dtype=jnp.float32)
    table = _merge_tables(neighbor_times.T, neighbor_ids.astype(jnp.int32).T,
                          eye)
    f1, f2, f3 = _sc_counts(src_ids.astype(jnp.int32),
                            dst_ids.astype(jnp.int32), interact_times, table)
    W1c = W1[2 * in_dim:3 * in_dim, :]
    W2c = Wf2 @ W1c
    b1p = b1 + bf2 @ W1c
    W1ab = W1[0:2 * in_dim, :]
    hpart = _mlp_a(input_1.T, input_2.T, W1ab, b1p)
    return _mlp_b(hpart, f1, f2, f3, Wf1, bf1, W2c, W2, b2).T
